# baseline (speedup 1.0000x reference)
import functools

import jax
import jax.numpy as jnp
from jax import lax
from jax.experimental import pallas as pl
from jax.experimental.pallas import tpu as pltpu
from jax.experimental.pallas import tpu_sc as plsc

N, P3, W3, CKV = 8, 49, 64, 384
TOPK = 4
ROWS = N * P3
B = N * P3 * TOPK
NC, NS = 2, 16
NW = NC * NS
KPB = 4
CHUNK = 49
MPAD = 224
PPAD = 64
NBUF = 4
L = 16

_mesh = plsc.VectorSubcoreMesh(core_axis_name="c", subcore_axis_name="s")


@functools.partial(
    pl.kernel,
    mesh=_mesh,
    out_type=jax.ShapeDtypeStruct((B, W3, CKV), jnp.float32),
    scratch_types=[
        pltpu.VMEM((MPAD,), jnp.int32),
        pltpu.VMEM((PPAD,), jnp.int32),
    ]
    + [pltpu.VMEM((1, W3, CKV), jnp.float32) for _ in range(NBUF)]
    + [pltpu.SemaphoreType.DMA for _ in range(2 * NBUF)],
)
def _sc_gather(meta_hbm, opl_hbm, table_hbm, out_hbm, mv, plv, *scr):
    bufs = scr[:NBUF]
    gsems = scr[NBUF : 2 * NBUF]
    wsems = scr[2 * NBUF :]
    wid = lax.axis_index("s") * NC + lax.axis_index("c")
    b = wid // KPB
    sbase = b * P3
    dbase = b * P3 * TOPK
    pltpu.sync_copy(meta_hbm.at[wid], mv)
    pltpu.sync_copy(opl_hbm.at[wid], plv)
    nr = mv[pl.ds(3 * PPAD, L)][0]

    def wait_one_write(u):
        pltpu.make_async_copy(bufs[u], out_hbm.at[pl.ds(0, 1)], wsems[u]).wait()

    def run_body(r, carry):
        rid_r = mv[pl.ds(r, L)][0]
        roff_r = mv[pl.ds(PPAD + r, L)][0]
        rcnt_r = mv[pl.ds(2 * PPAD + r, L)][0]
        s4 = lax.rem(r, NBUF)
        cnts = list(carry)
        for u in range(NBUF):

            @pl.when(s4 == u)
            def _(u=u):
                lax.fori_loop(
                    0, cnts[u], lambda t, c: (wait_one_write(u), c)[1], 0
                )
                pltpu.async_copy(
                    table_hbm.at[pl.ds(sbase + rid_r, 1)], bufs[u], gsems[u]
                )
                pltpu.make_async_copy(
                    table_hbm.at[pl.ds(sbase, 1)], bufs[u], gsems[u]
                ).wait()

                def wbody(t, c):
                    p = plv[pl.ds(roff_r + t, L)][0]
                    pltpu.async_copy(
                        bufs[u], out_hbm.at[pl.ds(dbase + p, 1)], wsems[u]
                    )
                    return c

                lax.fori_loop(0, rcnt_r, wbody, 0)

        return tuple(
            jnp.where(s4 == u, rcnt_r, cnts[u]) for u in range(NBUF)
        )

    carry = lax.fori_loop(0, nr, run_body, (jnp.int32(0),) * NBUF)

    for u in range(NBUF):
        lax.fori_loop(0, carry[u], lambda t, c, u=u: (wait_one_write(u), c)[1], 0)


def kernel(r_idx, qkv):
    ridx = r_idx.reshape(N, P3 * TOPK).astype(jnp.int32)
    order = jnp.argsort(ridx, axis=1, stable=True).astype(jnp.int32)
    sidx = jnp.take_along_axis(ridx, order, axis=1)
    chunk = sidx.reshape(NW, CHUNK)
    first = jnp.ones((NW, 1), bool)
    new = jnp.concatenate([first, chunk[:, 1:] != chunk[:, :-1]], axis=1)
    run_id = jnp.cumsum(new, axis=1, dtype=jnp.int32) - 1
    nr = jnp.sum(new, axis=1, dtype=jnp.int32)
    rids = jnp.arange(CHUNK, dtype=jnp.int32)
    onehot = run_id[:, :, None] == rids[None, None, :]
    pos = jnp.arange(CHUNK, dtype=jnp.int32)
    roff = jnp.argmax(onehot, axis=1).astype(jnp.int32)
    rcnt = jnp.sum(onehot, axis=1, dtype=jnp.int32)
    rid = jnp.take_along_axis(chunk, roff, axis=1)
    del pos
    meta = jnp.concatenate(
        [
            jnp.pad(rid, ((0, 0), (0, PPAD - CHUNK))),
            jnp.pad(roff, ((0, 0), (0, PPAD - CHUNK))),
            jnp.pad(rcnt, ((0, 0), (0, PPAD - CHUNK))),
            jnp.pad(nr[:, None], ((0, 0), (0, MPAD - 3 * PPAD - 1))),
        ],
        axis=1,
    )
    opl = jnp.pad(order.reshape(NW, CHUNK), ((0, 0), (0, PPAD - CHUNK)))
    table = qkv.reshape(ROWS, W3, CKV)
    out = _sc_gather(meta, opl, table)
    return out.reshape(N, P3, TOPK, W3, CKV)


# speedup vs baseline: 1.1255x; 1.1255x over previous
import functools

import jax
import jax.numpy as jnp
from jax import lax
from jax.experimental import pallas as pl
from jax.experimental.pallas import tpu as pltpu
from jax.experimental.pallas import tpu_sc as plsc

N, P3, W3, CKV = 8, 49, 64, 384
TOPK = 4
ROWS = N * P3
B = N * P3 * TOPK
NC, NS = 2, 16
NW = NC * NS
NQ = 4
WS = W3 // NQ
RPB = P3 * TOPK
PLPAD = 224
OFFPAD = 80
NBUF = 7
L = 16

_mesh = plsc.VectorSubcoreMesh(core_axis_name="c", subcore_axis_name="s")


@functools.partial(
    pl.kernel,
    mesh=_mesh,
    out_type=jax.ShapeDtypeStruct((B * NQ, WS, CKV), jnp.float32),
    scratch_types=[
        pltpu.VMEM((PLPAD,), jnp.int32),
        pltpu.VMEM((OFFPAD,), jnp.int32),
    ]
    + [pltpu.VMEM((1, WS, CKV), jnp.float32) for _ in range(NBUF)]
    + [pltpu.SemaphoreType.DMA for _ in range(2 * NBUF)],
)
def _sc_gather(plist_hbm, off_hbm, table_hbm, out_hbm, plv, offv, *scr):
    bufs = scr[:NBUF]
    gsems = scr[NBUF : 2 * NBUF]
    wsems = scr[2 * NBUF :]
    wid = lax.axis_index("s") * NC + lax.axis_index("c")
    b = wid // NQ
    q = wid % NQ
    sbase = b * RPB + q
    dbase = b * RPB * NQ + q

    def wait_one_write(u):
        pltpu.make_async_copy(bufs[u], out_hbm.at[pl.ds(0, 1)], wsems[u]).wait()

    def fire_gather(j, u):
        pltpu.async_copy(
            table_hbm.at[pl.ds(sbase + j * NQ, 1)], bufs[u], gsems[u]
        )

    def wait_gather(u):
        pltpu.make_async_copy(
            table_hbm.at[pl.ds(sbase, 1)], bufs[u], gsems[u]
        ).wait()

    def region(j, u, cnt_u2, last):
        o_j = offv[pl.ds(j, L)][0]
        c_j = offv[pl.ds(j + 1, L)][0] - o_j
        wait_gather(u)
        if not last:
            u2 = (u + 2) % NBUF
            lax.fori_loop(0, cnt_u2, lambda t, c: (wait_one_write(u2), c)[1], 0)
            fire_gather(j + 2, u2)

        def wbody(t, c):
            p = plv[pl.ds(o_j + t, L)][0]
            pltpu.async_copy(
                bufs[u], out_hbm.at[pl.ds(dbase + p * NQ, 1)], wsems[u]
            )
            return c

        lax.fori_loop(0, c_j, wbody, 0)
        return c_j

    fire_gather(0, 0)
    fire_gather(1, 1)
    pltpu.sync_copy(plist_hbm.at[b], plv)
    pltpu.sync_copy(off_hbm.at[b], offv)

    def block(g, carry):
        cnts = list(carry)
        for u in range(NBUF):
            cnts[u] = region(NBUF * g + u, u, cnts[(u + 2) % NBUF], False)
        return tuple(cnts)

    carry = lax.fori_loop(0, P3 // NBUF - 1, block, (jnp.int32(0),) * NBUF)

    cnts = list(carry)
    for u in range(NBUF):
        j = P3 - NBUF + u
        cnts[u] = region(j, u, cnts[(u + 2) % NBUF], j >= P3 - 2)

    for u in range(NBUF):
        lax.fori_loop(0, cnts[u], lambda t, c, u=u: (wait_one_write(u), c)[1], 0)


def kernel(r_idx, qkv):
    ridx = r_idx.reshape(N, RPB).astype(jnp.int32)
    order = jnp.argsort(ridx, axis=1, stable=True).astype(jnp.int32)
    counts = jnp.sum(
        ridx[:, :, None] == jnp.arange(P3, dtype=jnp.int32)[None, None, :],
        axis=1,
        dtype=jnp.int32,
    )
    offsets = jnp.concatenate(
        [jnp.zeros((N, 1), jnp.int32), jnp.cumsum(counts, axis=1, dtype=jnp.int32)],
        axis=1,
    )
    plist = jnp.pad(order, ((0, 0), (0, PLPAD - RPB)))
    offs = jnp.pad(offsets, ((0, 0), (0, OFFPAD - (P3 + 1))))
    table = qkv.reshape(ROWS * NQ, WS, CKV)
    out = _sc_gather(plist, offs, table)
    return out.reshape(N, P3, TOPK, W3, CKV)


# speedup vs baseline: 1.1258x; 1.0003x over previous
import functools

import jax
import jax.numpy as jnp
from jax import lax
from jax.experimental import pallas as pl
from jax.experimental.pallas import tpu as pltpu
from jax.experimental.pallas import tpu_sc as plsc

N, P3, W3, CKV = 8, 49, 64, 384
TOPK = 4
ROWS = N * P3
B = N * P3 * TOPK
NC, NS = 2, 16
NW = NC * NS
NQ = 4
WS = W3 // NQ
RPB = P3 * TOPK
PLPAD = 224
OFFPAD = 80
NBUF = 7
L = 16

_mesh = plsc.VectorSubcoreMesh(core_axis_name="c", subcore_axis_name="s")


@functools.partial(
    pl.kernel,
    mesh=_mesh,
    out_type=jax.ShapeDtypeStruct((B * NQ, WS, CKV), jnp.float32),
    scratch_types=[
        pltpu.VMEM((PLPAD,), jnp.int32),
        pltpu.VMEM((OFFPAD,), jnp.int32),
    ]
    + [pltpu.VMEM((1, WS, CKV), jnp.float32) for _ in range(NBUF)]
    + [pltpu.SemaphoreType.DMA for _ in range(2 * NBUF)],
)
def _sc_gather(plist_hbm, off_hbm, table_hbm, out_hbm, plv, offv, *scr):
    bufs = scr[:NBUF]
    gsems = scr[NBUF : 2 * NBUF]
    wsems = scr[2 * NBUF :]
    wid = lax.axis_index("s") * NC + lax.axis_index("c")
    b = wid // NQ
    q = wid % NQ
    sbase = b * RPB + q
    dbase = b * RPB * NQ + q

    def wait_one_write(u):
        pltpu.make_async_copy(bufs[u], out_hbm.at[pl.ds(0, 1)], wsems[u]).wait()

    def fire_gather(j, u):
        pltpu.async_copy(
            table_hbm.at[pl.ds(sbase + j * NQ, 1)], bufs[u], gsems[u]
        )

    def wait_gather(u):
        pltpu.make_async_copy(
            table_hbm.at[pl.ds(sbase, 1)], bufs[u], gsems[u]
        ).wait()

    def region(j, u, cnt_u2, last):
        o_j = offv[pl.ds(j, L)][0]
        c_j = offv[pl.ds(j + 1, L)][0] - o_j
        wait_gather(u)
        if not last:
            u2 = (u + 3) % NBUF
            lax.fori_loop(0, cnt_u2, lambda t, c: (wait_one_write(u2), c)[1], 0)
            fire_gather(j + 3, u2)

        def wbody(t, c):
            p = plv[pl.ds(o_j + t, L)][0]
            pltpu.async_copy(
                bufs[u], out_hbm.at[pl.ds(dbase + p * NQ, 1)], wsems[u]
            )
            return c

        lax.fori_loop(0, c_j, wbody, 0)
        return c_j

    fire_gather(0, 0)
    fire_gather(1, 1)
    fire_gather(2, 2)
    pltpu.sync_copy(plist_hbm.at[b], plv)
    pltpu.sync_copy(off_hbm.at[b], offv)

    def block(g, carry):
        cnts = list(carry)
        for u in range(NBUF):
            cnts[u] = region(NBUF * g + u, u, cnts[(u + 3) % NBUF], False)
        return tuple(cnts)

    carry = lax.fori_loop(0, P3 // NBUF - 1, block, (jnp.int32(0),) * NBUF)

    cnts = list(carry)
    for u in range(NBUF):
        j = P3 - NBUF + u
        cnts[u] = region(j, u, cnts[(u + 3) % NBUF], j >= P3 - 3)

    for u in range(NBUF):
        lax.fori_loop(0, cnts[u], lambda t, c, u=u: (wait_one_write(u), c)[1], 0)


def kernel(r_idx, qkv):
    ridx = r_idx.reshape(N, RPB).astype(jnp.int32)
    order = jnp.argsort(ridx, axis=1, stable=True).astype(jnp.int32)
    counts = jnp.sum(
        ridx[:, :, None] == jnp.arange(P3, dtype=jnp.int32)[None, None, :],
        axis=1,
        dtype=jnp.int32,
    )
    offsets = jnp.concatenate(
        [jnp.zeros((N, 1), jnp.int32), jnp.cumsum(counts, axis=1, dtype=jnp.int32)],
        axis=1,
    )
    plist = jnp.pad(order, ((0, 0), (0, PLPAD - RPB)))
    offs = jnp.pad(offsets, ((0, 0), (0, OFFPAD - (P3 + 1))))
    table = qkv.reshape(ROWS * NQ, WS, CKV)
    out = _sc_gather(plist, offs, table)
    return out.reshape(N, P3, TOPK, W3, CKV)
